# baseline (speedup 1.0000x reference)
import functools

import jax
import jax.numpy as jnp
from jax import lax
from jax.experimental import pallas as pl
from jax.experimental.pallas import tpu as pltpu
from jax.experimental.pallas import tpu_sc as plsc

N = 94
NP = 96
E = 3008
L = 16
EV = E // L
NV = NP // L
UNROLL = 4


def _splat(ref, k):
    return plsc.load_gather(ref, [jnp.full((L,), k, jnp.int32)])


def _rsqrt16(x):
    i = plsc.bitcast(x, jnp.int32)
    i = jnp.int32(0x5F3759DF) - (i >> 1)
    y = plsc.bitcast(i, jnp.float32)
    for _ in range(3):
        y = y * (1.5 - 0.5 * x * y * y)
    return y


PBASE = 288


def _gcn_body(fh, eh, outh,
              fv, ev,
              deg, dinv, xa, xb, ga, gb, xc, gc, normv, hout):
    cid = lax.axis_index("c")
    sid = lax.axis_index("s")

    @pl.when(jnp.logical_and(cid == 0, sid == 0))
    def _():
        pltpu.sync_copy(fh, fv)
        pltpu.sync_copy(eh, ev)
        params = fv

        zeros = jnp.zeros((L,), jnp.float32)
        ones = jnp.full((L,), 1.0, jnp.float32)

        for i in range(NV):
            deg[pl.ds(i * L, L)] = zeros

        def degbody(i, c):
            for u in range(UNROLL):
                off = pl.multiple_of(i * (L * UNROLL) + u * L, L)
                plsc.addupdate_scatter(deg, [ev[pl.ds(E + off, L)]], ones)
            return c
        lax.fori_loop(0, EV // UNROLL, degbody, 0)

        for i in range(NV):
            s = pl.ds(i * L, L)
            idx = lax.iota(jnp.int32, L) + i * L
            real = idx < N
            dg = deg[s] + jnp.where(real, 1.0, 0.0)
            dinv[s] = _rsqrt16(jnp.where(real, dg, 1.0))

        w00 = _splat(params, PBASE + 0)
        w01 = _splat(params, PBASE + 1)
        w10 = _splat(params, PBASE + 2)
        w11 = _splat(params, PBASE + 3)
        w20 = _splat(params, PBASE + 4)
        w21 = _splat(params, PBASE + 5)
        for i in range(NV):
            s = pl.ds(i * L, L)
            base = jnp.minimum(lax.iota(jnp.int32, L) + i * L, N - 1) * 3
            f0 = plsc.load_gather(fv, [base])
            f1 = plsc.load_gather(fv, [base + 1])
            f2 = plsc.load_gather(fv, [base + 2])
            xa[s] = f0 * w00 + f1 * w10 + f2 * w20
            xb[s] = f0 * w01 + f1 * w11 + f2 * w21
            ga[s] = zeros
            gb[s] = zeros

        def e1(i, c):
            for u in range(UNROLL):
                off = pl.multiple_of(i * (L * UNROLL) + u * L, L)
                s = pl.ds(off, L)
                sv = ev[s]
                dv = ev[pl.ds(E + off, L)]
                nm = plsc.load_gather(dinv, [sv]) * plsc.load_gather(dinv, [dv])
                normv[s] = nm
                plsc.addupdate_scatter(ga, [dv], nm * plsc.load_gather(xa, [sv]))
                plsc.addupdate_scatter(gb, [dv], nm * plsc.load_gather(xb, [sv]))
            return c
        lax.fori_loop(0, EV // UNROLL, e1, 0)

        b1a = _splat(params, PBASE + 6)
        b1b = _splat(params, PBASE + 7)
        w2a = _splat(params, PBASE + 8)
        w2b = _splat(params, PBASE + 9)
        for i in range(NV):
            s = pl.ds(i * L, L)
            dv2 = dinv[s] * dinv[s]
            va = jnp.maximum(ga[s] + dv2 * xa[s] + b1a, 0.0)
            vb = jnp.maximum(gb[s] + dv2 * xb[s] + b1b, 0.0)
            xc[s] = va * w2a + vb * w2b
            gc[s] = zeros

        def e2(i, c):
            for u in range(UNROLL):
                off = pl.multiple_of(i * (L * UNROLL) + u * L, L)
                s = pl.ds(off, L)
                sv = ev[s]
                dv = ev[pl.ds(E + off, L)]
                plsc.addupdate_scatter(gc, [dv],
                                       normv[s] * plsc.load_gather(xc, [sv]))
            return c
        lax.fori_loop(0, EV // UNROLL, e2, 0)

        b2s = _splat(params, PBASE + 10)
        for i in range(NV):
            s = pl.ds(i * L, L)
            idx = lax.iota(jnp.int32, L) + i * L
            dv2 = dinv[s] * dinv[s]
            hv = jnp.maximum(gc[s] + dv2 * xc[s] + b2s, 0.0)
            hout[pl.ds(i * L, L)] = jnp.where(idx < N, hv, 0.0)

        pltpu.sync_copy(hout, outh)


_SC_SCRATCH = [
    pltpu.VMEM((304,), jnp.float32),
    pltpu.VMEM((2 * E,), jnp.int32),
    pltpu.VMEM((NP,), jnp.float32),
    pltpu.VMEM((NP,), jnp.float32),
    pltpu.VMEM((NP,), jnp.float32),
    pltpu.VMEM((NP,), jnp.float32),
    pltpu.VMEM((NP,), jnp.float32),
    pltpu.VMEM((NP,), jnp.float32),
    pltpu.VMEM((NP,), jnp.float32),
    pltpu.VMEM((NP,), jnp.float32),
    pltpu.VMEM((E,), jnp.float32),
    pltpu.VMEM((NP,), jnp.float32),
]


def _sc_gcn(fp, ef):
    fn = functools.partial(
        pl.kernel,
        out_type=jax.ShapeDtypeStruct((NP,), jnp.float32),
        mesh=plsc.VectorSubcoreMesh(core_axis_name="c", subcore_axis_name="s"),
        scratch_types=_SC_SCRATCH,
        compiler_params=pltpu.CompilerParams(needs_layout_passes=False),
    )(_gcn_body)
    return fn(fp, ef)


KBLK = 128
NBLK = 1024 // KBLK


def _tc_body(h_ref, w1_ref, b1_ref, w2_ref, b2_ref, w3_ref, b3_ref,
             o_ref, h2_ref):
    i = pl.program_id(0)

    @pl.when(i == 0)
    def _():
        x = h_ref[...]
        w1 = jnp.concatenate(
            [w1_ref[...], jnp.zeros((NP - N, 512), jnp.float32)])
        h1 = jnp.sum(x * w1, axis=0, keepdims=True)
        h1 = jnp.maximum(h1 + b1_ref[...], 0.0)
        h2 = lax.dot_general(w2_ref[...], h1,
                             (((0,), (1,)), ((), ())),
                             precision=lax.Precision.HIGHEST,
                             preferred_element_type=jnp.float32)
        h2_ref[...] = jnp.maximum(h2 + b2_ref[...], 0.0)

    acc = h2_ref[pl.ds(i * KBLK, KBLK), :] * w3_ref[...]
    for sz in (64, 32, 16, 8):
        acc = acc[:sz] + acc[sz:]
    part = jnp.sum(acc, axis=0, keepdims=True)

    @pl.when(i == 0)
    def _():
        o_ref[...] = part + b3_ref[...]

    @pl.when(i > 0)
    def _():
        o_ref[...] += part


def _tc_mlp(hcol, w1, b1r, w2, b2c, w3, b3r):
    return pl.pallas_call(
        _tc_body,
        grid=(NBLK,),
        in_specs=[
            pl.BlockSpec((NP, 1), lambda j: (0, 0)),
            pl.BlockSpec((N, 512), lambda j: (0, 0)),
            pl.BlockSpec((1, 512), lambda j: (0, 0)),
            pl.BlockSpec((512, 1024), lambda j: (0, 0)),
            pl.BlockSpec((1024, 1), lambda j: (0, 0)),
            pl.BlockSpec((KBLK, 6400), lambda j: (j, 0)),
            pl.BlockSpec((1, 6400), lambda j: (0, 0)),
        ],
        out_specs=pl.BlockSpec((1, 6400), lambda j: (0, 0)),
        out_shape=jax.ShapeDtypeStruct((1, 6400), jnp.float32),
        scratch_shapes=[pltpu.VMEM((1024, 1), jnp.float32)],
        compiler_params=pltpu.CompilerParams(
            dimension_semantics=("arbitrary",)),
    )(hcol, w1, b1r, w2, b2c, w3, b3r)


def kernel(feature, edge_index, W1, b1, W2, b2, Wfc1, bfc1, Wfc2, bfc2, Wfc, bfc):
    fp = jnp.concatenate(
        [feature.reshape(-1), jnp.zeros((PBASE - 3 * N,), jnp.float32),
         W1.reshape(-1), b1, W2.reshape(-1), b2,
         jnp.zeros((5,), jnp.float32)])
    h96 = _sc_gcn(fp, edge_index.reshape(-1))
    out = _tc_mlp(h96.reshape(NP, 1),
                  Wfc1,
                  bfc1.reshape(1, -1),
                  Wfc2,
                  bfc2.reshape(-1, 1),
                  Wfc,
                  bfc.reshape(1, -1))
    return out.reshape(-1)


# speedup vs baseline: 1.1261x; 1.1261x over previous
import functools

import jax
import jax.numpy as jnp
from jax import lax
from jax.experimental import pallas as pl
from jax.experimental.pallas import tpu as pltpu
from jax.experimental.pallas import tpu_sc as plsc

N = 94
NP = 96
E = 3008
L = 16
EV = E // L
NV = NP // L
UNROLL = 4
PBASE = 288
EBASE = 304
PKLEN = EBASE + 2 * E


def _splat(ref, k):
    return plsc.load_gather(ref, [jnp.full((L,), k, jnp.int32)])


def _rsqrt16(x):
    i = plsc.bitcast(x, jnp.int32)
    i = jnp.int32(0x5F3759DF) - (i >> 1)
    y = plsc.bitcast(i, jnp.float32)
    for _ in range(3):
        y = y * (1.5 - 0.5 * x * y * y)
    return y


def _gcn_body(pkh, outh,
              pk,
              deg, dinv, xa, xb, ga, gb, xc, gc, normv, hout):
    cid = lax.axis_index("c")
    sid = lax.axis_index("s")

    @pl.when(jnp.logical_and(cid == 0, sid == 0))
    def _():
        pltpu.sync_copy(pkh, pk)

        def src16(off):
            return plsc.bitcast(pk[pl.ds(EBASE + off, L)], jnp.int32)

        def dst16(off):
            return plsc.bitcast(pk[pl.ds(EBASE + E + off, L)], jnp.int32)

        zeros = jnp.zeros((L,), jnp.float32)
        ones = jnp.full((L,), 1.0, jnp.float32)

        for i in range(NV):
            deg[pl.ds(i * L, L)] = zeros

        def degbody(i, c):
            for u in range(UNROLL):
                off = pl.multiple_of(i * (L * UNROLL) + u * L, L)
                plsc.addupdate_scatter(deg, [dst16(off)], ones)
            return c
        lax.fori_loop(0, EV // UNROLL, degbody, 0)

        for i in range(NV):
            s = pl.ds(i * L, L)
            idx = lax.iota(jnp.int32, L) + i * L
            real = idx < N
            dg = deg[s] + jnp.where(real, 1.0, 0.0)
            dinv[s] = _rsqrt16(jnp.where(real, dg, 1.0))

        w00 = _splat(pk, PBASE + 0)
        w01 = _splat(pk, PBASE + 1)
        w10 = _splat(pk, PBASE + 2)
        w11 = _splat(pk, PBASE + 3)
        w20 = _splat(pk, PBASE + 4)
        w21 = _splat(pk, PBASE + 5)
        for i in range(NV):
            s = pl.ds(i * L, L)
            base = jnp.minimum(lax.iota(jnp.int32, L) + i * L, N - 1) * 3
            f0 = plsc.load_gather(pk, [base])
            f1 = plsc.load_gather(pk, [base + 1])
            f2 = plsc.load_gather(pk, [base + 2])
            xa[s] = f0 * w00 + f1 * w10 + f2 * w20
            xb[s] = f0 * w01 + f1 * w11 + f2 * w21
            ga[s] = zeros
            gb[s] = zeros

        def e1(i, c):
            for u in range(UNROLL):
                off = pl.multiple_of(i * (L * UNROLL) + u * L, L)
                sv = src16(off)
                dv = dst16(off)
                nm = plsc.load_gather(dinv, [sv]) * plsc.load_gather(dinv, [dv])
                normv[pl.ds(off, L)] = nm
                plsc.addupdate_scatter(ga, [dv], nm * plsc.load_gather(xa, [sv]))
                plsc.addupdate_scatter(gb, [dv], nm * plsc.load_gather(xb, [sv]))
            return c
        lax.fori_loop(0, EV // UNROLL, e1, 0)

        b1a = _splat(pk, PBASE + 6)
        b1b = _splat(pk, PBASE + 7)
        w2a = _splat(pk, PBASE + 8)
        w2b = _splat(pk, PBASE + 9)
        for i in range(NV):
            s = pl.ds(i * L, L)
            dv2 = dinv[s] * dinv[s]
            va = jnp.maximum(ga[s] + dv2 * xa[s] + b1a, 0.0)
            vb = jnp.maximum(gb[s] + dv2 * xb[s] + b1b, 0.0)
            xc[s] = va * w2a + vb * w2b
            gc[s] = zeros

        def e2(i, c):
            for u in range(UNROLL):
                off = pl.multiple_of(i * (L * UNROLL) + u * L, L)
                sv = src16(off)
                dv = dst16(off)
                plsc.addupdate_scatter(gc, [dv],
                                       normv[pl.ds(off, L)] * plsc.load_gather(xc, [sv]))
            return c
        lax.fori_loop(0, EV // UNROLL, e2, 0)

        b2s = _splat(pk, PBASE + 10)
        for i in range(NV):
            s = pl.ds(i * L, L)
            idx = lax.iota(jnp.int32, L) + i * L
            dv2 = dinv[s] * dinv[s]
            hv = jnp.maximum(gc[s] + dv2 * xc[s] + b2s, 0.0)
            hout[pl.ds(i * L, L)] = jnp.where(idx < N, hv, 0.0)

        pltpu.sync_copy(hout, outh)


_SC_SCRATCH = [
    pltpu.VMEM((PKLEN,), jnp.float32),
    pltpu.VMEM((NP,), jnp.float32),
    pltpu.VMEM((NP,), jnp.float32),
    pltpu.VMEM((NP,), jnp.float32),
    pltpu.VMEM((NP,), jnp.float32),
    pltpu.VMEM((NP,), jnp.float32),
    pltpu.VMEM((NP,), jnp.float32),
    pltpu.VMEM((NP,), jnp.float32),
    pltpu.VMEM((NP,), jnp.float32),
    pltpu.VMEM((E,), jnp.float32),
    pltpu.VMEM((NP,), jnp.float32),
]


def _sc_gcn(pk):
    fn = functools.partial(
        pl.kernel,
        out_type=jax.ShapeDtypeStruct((NP,), jnp.float32),
        mesh=plsc.VectorSubcoreMesh(core_axis_name="c", subcore_axis_name="s"),
        scratch_types=_SC_SCRATCH,
        compiler_params=pltpu.CompilerParams(needs_layout_passes=False),
    )(_gcn_body)
    return fn(pk)


KBLK = 128
NBLK = 1024 // KBLK


def _tc_body(h_ref, w1_ref, b1_ref, w2_ref, b2_ref, w3_ref, b3_ref,
             o_ref, h2_ref):
    i = pl.program_id(0)

    @pl.when(i == 0)
    def _():
        x = jnp.transpose(h_ref[...][None, :], (1, 0))
        w1 = jnp.concatenate(
            [w1_ref[...], jnp.zeros((NP - N, 512), jnp.float32)])
        h1 = jnp.sum(x * w1, axis=0, keepdims=True)
        h1 = jnp.maximum(h1 + b1_ref[...][None, :], 0.0)
        h1c = jnp.transpose(h1, (1, 0))
        h2 = jnp.sum(h1c * w2_ref[...], axis=0, keepdims=True)
        h2 = jnp.maximum(h2 + b2_ref[...][None, :], 0.0)
        h2_ref[...] = jnp.transpose(h2, (1, 0))

    acc = h2_ref[pl.ds(i * KBLK, KBLK), :] * w3_ref[...]
    for sz in (64, 32, 16, 8):
        acc = acc[:sz] + acc[sz:]
    part = jnp.sum(acc, axis=0)

    @pl.when(i == 0)
    def _():
        o_ref[...] = part + b3_ref[...]

    @pl.when(i > 0)
    def _():
        o_ref[...] += part


def _tc_mlp(h, w1, b1, w2, b2, w3, b3):
    return pl.pallas_call(
        _tc_body,
        grid=(NBLK,),
        in_specs=[
            pl.BlockSpec((NP,), lambda j: (0,)),
            pl.BlockSpec((N, 512), lambda j: (0, 0)),
            pl.BlockSpec((512,), lambda j: (0,)),
            pl.BlockSpec((512, 1024), lambda j: (0, 0)),
            pl.BlockSpec((1024,), lambda j: (0,)),
            pl.BlockSpec((KBLK, 6400), lambda j: (j, 0)),
            pl.BlockSpec((6400,), lambda j: (0,)),
        ],
        out_specs=pl.BlockSpec((6400,), lambda j: (0,)),
        out_shape=jax.ShapeDtypeStruct((6400,), jnp.float32),
        scratch_shapes=[pltpu.VMEM((1024, 1), jnp.float32)],
        compiler_params=pltpu.CompilerParams(
            dimension_semantics=("arbitrary",)),
    )(h, w1, b1, w2, b2, w3, b3)


def kernel(feature, edge_index, W1, b1, W2, b2, Wfc1, bfc1, Wfc2, bfc2, Wfc, bfc):
    pk = jnp.concatenate(
        [feature.reshape(-1), jnp.zeros((PBASE - 3 * N,), jnp.float32),
         W1.reshape(-1), b1, W2.reshape(-1), b2,
         jnp.zeros((EBASE - PBASE - 11,), jnp.float32),
         lax.bitcast_convert_type(edge_index.reshape(-1), jnp.float32)])
    h96 = _sc_gcn(pk)
    return _tc_mlp(h96, Wfc1, bfc1, Wfc2, bfc2, Wfc, bfc)


# speedup vs baseline: 1.2769x; 1.1340x over previous
import functools

import jax
import jax.numpy as jnp
from jax import lax
from jax.experimental import pallas as pl
from jax.experimental.pallas import tpu as pltpu
from jax.experimental.pallas import tpu_sc as plsc

N = 94
NP = 96
E = 3008
L = 16
EV = E // L
NV = NP // L
UNROLL = 4
PBASE = 288
EBASE = 304
PKLEN = EBASE + 2 * E


def _splat(ref, k):
    return plsc.load_gather(ref, [jnp.full((L,), k, jnp.int32)])


def _rsqrt16(x):
    i = plsc.bitcast(x, jnp.int32)
    i = jnp.int32(0x5F3759DF) - (i >> 1)
    y = plsc.bitcast(i, jnp.float32)
    for _ in range(3):
        y = y * (1.5 - 0.5 * x * y * y)
    return y


GPT = 12


def _gcn_body(pkh, outh,
              pk,
              deg, dinv, xa, xb, ga, gb, xc, gc, normv, hout, zbuf, iref,
              sdeg, sga, sgb, sgc):
    cid = lax.axis_index("c")
    sid = lax.axis_index("s")

    @pl.when(cid == 0)
    def _():
        pltpu.sync_copy(pkh, pk)

        def src16(off):
            return plsc.bitcast(pk[pl.ds(EBASE + off, L)], jnp.int32)

        def dst16(off):
            return plsc.bitcast(pk[pl.ds(EBASE + E + off, L)], jnp.int32)

        zeros = jnp.zeros((L,), jnp.float32)
        ones = jnp.full((L,), 1.0, jnp.float32)
        for i in range(NV):
            zbuf[pl.ds(i * L, L)] = zeros
            iref[pl.ds(i * L, L)] = lax.iota(jnp.int32, L) + i * L

        @pl.when(sid == 0)
        def _():
            pltpu.sync_copy(zbuf, sdeg)
            pltpu.sync_copy(zbuf, sga)
            pltpu.sync_copy(zbuf, sgb)
            pltpu.sync_copy(zbuf, sgc)

        for i in range(NV):
            deg[pl.ds(i * L, L)] = zeros

        def degbody(i, c):
            g = sid + i * L
            @pl.when(g < EV)
            def _():
                off = pl.multiple_of(g * L, L)
                plsc.addupdate_scatter(deg, [dst16(off)], ones)
            return c
        lax.fori_loop(0, GPT, degbody, 0)

        plsc.subcore_barrier()
        pltpu.sync_copy(deg, sdeg.at[iref], add=True)
        plsc.subcore_barrier()
        pltpu.sync_copy(sdeg, deg)

        for i in range(NV):
            s = pl.ds(i * L, L)
            idx = lax.iota(jnp.int32, L) + i * L
            real = idx < N
            dg = deg[s] + jnp.where(real, 1.0, 0.0)
            dinv[s] = _rsqrt16(jnp.where(real, dg, 1.0))

        w00 = _splat(pk, PBASE + 0)
        w01 = _splat(pk, PBASE + 1)
        w10 = _splat(pk, PBASE + 2)
        w11 = _splat(pk, PBASE + 3)
        w20 = _splat(pk, PBASE + 4)
        w21 = _splat(pk, PBASE + 5)
        for i in range(NV):
            s = pl.ds(i * L, L)
            base = jnp.minimum(lax.iota(jnp.int32, L) + i * L, N - 1) * 3
            f0 = plsc.load_gather(pk, [base])
            f1 = plsc.load_gather(pk, [base + 1])
            f2 = plsc.load_gather(pk, [base + 2])
            xa[s] = f0 * w00 + f1 * w10 + f2 * w20
            xb[s] = f0 * w01 + f1 * w11 + f2 * w21
            ga[s] = zeros
            gb[s] = zeros

        def e1(i, c):
            g = sid + i * L
            @pl.when(g < EV)
            def _():
                off = pl.multiple_of(g * L, L)
                sv = src16(off)
                dv = dst16(off)
                nm = plsc.load_gather(dinv, [sv]) * plsc.load_gather(dinv, [dv])
                normv[pl.ds(off, L)] = nm
                plsc.addupdate_scatter(ga, [dv], nm * plsc.load_gather(xa, [sv]))
                plsc.addupdate_scatter(gb, [dv], nm * plsc.load_gather(xb, [sv]))
            return c
        lax.fori_loop(0, GPT, e1, 0)

        pltpu.sync_copy(ga, sga.at[iref], add=True)
        pltpu.sync_copy(gb, sgb.at[iref], add=True)
        plsc.subcore_barrier()
        pltpu.sync_copy(sga, ga)
        pltpu.sync_copy(sgb, gb)

        b1a = _splat(pk, PBASE + 6)
        b1b = _splat(pk, PBASE + 7)
        w2a = _splat(pk, PBASE + 8)
        w2b = _splat(pk, PBASE + 9)
        for i in range(NV):
            s = pl.ds(i * L, L)
            dv2 = dinv[s] * dinv[s]
            va = jnp.maximum(ga[s] + dv2 * xa[s] + b1a, 0.0)
            vb = jnp.maximum(gb[s] + dv2 * xb[s] + b1b, 0.0)
            xc[s] = va * w2a + vb * w2b
            gc[s] = zeros

        def e2(i, c):
            g = sid + i * L
            @pl.when(g < EV)
            def _():
                off = pl.multiple_of(g * L, L)
                sv = src16(off)
                dv = dst16(off)
                plsc.addupdate_scatter(gc, [dv],
                                       normv[pl.ds(off, L)] * plsc.load_gather(xc, [sv]))
            return c
        lax.fori_loop(0, GPT, e2, 0)

        pltpu.sync_copy(gc, sgc.at[iref], add=True)
        plsc.subcore_barrier()

        @pl.when(sid == 0)
        def _():
            pltpu.sync_copy(sgc, gc)
            b2s = _splat(pk, PBASE + 10)
            for i in range(NV):
                s = pl.ds(i * L, L)
                idx = lax.iota(jnp.int32, L) + i * L
                dv2 = dinv[s] * dinv[s]
                hv = jnp.maximum(gc[s] + dv2 * xc[s] + b2s, 0.0)
                hout[pl.ds(i * L, L)] = jnp.where(idx < N, hv, 0.0)
            pltpu.sync_copy(hout, outh)


_SC_SCRATCH = [
    pltpu.VMEM((PKLEN,), jnp.float32),
    pltpu.VMEM((NP,), jnp.float32),
    pltpu.VMEM((NP,), jnp.float32),
    pltpu.VMEM((NP,), jnp.float32),
    pltpu.VMEM((NP,), jnp.float32),
    pltpu.VMEM((NP,), jnp.float32),
    pltpu.VMEM((NP,), jnp.float32),
    pltpu.VMEM((NP,), jnp.float32),
    pltpu.VMEM((NP,), jnp.float32),
    pltpu.VMEM((E,), jnp.float32),
    pltpu.VMEM((NP,), jnp.float32),
    pltpu.VMEM((NP,), jnp.float32),
    pltpu.VMEM((NP,), jnp.int32),
    pltpu.VMEM_SHARED((NP,), jnp.float32),
    pltpu.VMEM_SHARED((NP,), jnp.float32),
    pltpu.VMEM_SHARED((NP,), jnp.float32),
    pltpu.VMEM_SHARED((NP,), jnp.float32),
]


def _sc_gcn(pk):
    fn = functools.partial(
        pl.kernel,
        out_type=jax.ShapeDtypeStruct((NP,), jnp.float32),
        mesh=plsc.VectorSubcoreMesh(core_axis_name="c", subcore_axis_name="s"),
        scratch_types=_SC_SCRATCH,
        compiler_params=pltpu.CompilerParams(needs_layout_passes=False),
    )(_gcn_body)
    return fn(pk)


KBLK = 128
NBLK = 1024 // KBLK


def _tc_body(h_ref, w1_ref, b1_ref, w2_ref, b2_ref, w3_ref, b3_ref,
             o_ref, h2_ref):
    i = pl.program_id(0)

    @pl.when(i == 0)
    def _():
        x = jnp.transpose(h_ref[...][None, :], (1, 0))
        w1 = jnp.concatenate(
            [w1_ref[...], jnp.zeros((NP - N, 512), jnp.float32)])
        h1 = jnp.sum(x * w1, axis=0, keepdims=True)
        h1 = jnp.maximum(h1 + b1_ref[...][None, :], 0.0)
        h1c = jnp.transpose(h1, (1, 0))
        h2 = jnp.sum(h1c * w2_ref[...], axis=0, keepdims=True)
        h2 = jnp.maximum(h2 + b2_ref[...][None, :], 0.0)
        h2_ref[...] = jnp.transpose(h2, (1, 0))

    acc = h2_ref[pl.ds(i * KBLK, KBLK), :] * w3_ref[...]
    for sz in (64, 32, 16, 8):
        acc = acc[:sz] + acc[sz:]
    part = jnp.sum(acc, axis=0)

    @pl.when(i == 0)
    def _():
        o_ref[...] = part + b3_ref[...]

    @pl.when(i > 0)
    def _():
        o_ref[...] += part


def _tc_mlp(h, w1, b1, w2, b2, w3, b3):
    return pl.pallas_call(
        _tc_body,
        grid=(NBLK,),
        in_specs=[
            pl.BlockSpec((NP,), lambda j: (0,)),
            pl.BlockSpec((N, 512), lambda j: (0, 0)),
            pl.BlockSpec((512,), lambda j: (0,)),
            pl.BlockSpec((512, 1024), lambda j: (0, 0)),
            pl.BlockSpec((1024,), lambda j: (0,)),
            pl.BlockSpec((KBLK, 6400), lambda j: (j, 0)),
            pl.BlockSpec((6400,), lambda j: (0,)),
        ],
        out_specs=pl.BlockSpec((6400,), lambda j: (0,)),
        out_shape=jax.ShapeDtypeStruct((6400,), jnp.float32),
        scratch_shapes=[pltpu.VMEM((1024, 1), jnp.float32)],
        compiler_params=pltpu.CompilerParams(
            dimension_semantics=("arbitrary",)),
    )(h, w1, b1, w2, b2, w3, b3)


def kernel(feature, edge_index, W1, b1, W2, b2, Wfc1, bfc1, Wfc2, bfc2, Wfc, bfc):
    pk = jnp.concatenate(
        [feature.reshape(-1), jnp.zeros((PBASE - 3 * N,), jnp.float32),
         W1.reshape(-1), b1, W2.reshape(-1), b2,
         jnp.zeros((EBASE - PBASE - 11,), jnp.float32),
         lax.bitcast_convert_type(edge_index.reshape(-1), jnp.float32)])
    h96 = _sc_gcn(pk)
    return _tc_mlp(h96, Wfc1, bfc1, Wfc2, bfc2, Wfc, bfc)


# speedup vs baseline: 1.3280x; 1.0400x over previous
import functools

import jax
import jax.numpy as jnp
from jax import lax
from jax.experimental import pallas as pl
from jax.experimental.pallas import tpu as pltpu
from jax.experimental.pallas import tpu_sc as plsc

N = 94
NP = 96
E = 3008
L = 16
EV = E // L
NV = NP // L
UNROLL = 4
PBASE = 288
EBASE = 304
PKLEN = EBASE + 2 * E


def _splat(ref, k):
    return plsc.load_gather(ref, [jnp.full((L,), k, jnp.int32)])


def _rsqrt16(x):
    i = plsc.bitcast(x, jnp.int32)
    i = jnp.int32(0x5F3759DF) - (i >> 1)
    y = plsc.bitcast(i, jnp.float32)
    for _ in range(3):
        y = y * (1.5 - 0.5 * x * y * y)
    return y


GPT = 12


def _gcn_body(pkh, outh,
              pk,
              deg, dinv, xa, xb, ga, gb, xc, gc, normv, hout, zbuf, iref,
              sdeg, sga, sgb, sgc):
    cid = lax.axis_index("c")
    sid = lax.axis_index("s")

    @pl.when(cid == 0)
    def _():
        pltpu.sync_copy(pkh, pk)

        def src16(off):
            return plsc.bitcast(pk[pl.ds(EBASE + off, L)], jnp.int32)

        def dst16(off):
            return plsc.bitcast(pk[pl.ds(EBASE + E + off, L)], jnp.int32)

        zeros = jnp.zeros((L,), jnp.float32)
        ones = jnp.full((L,), 1.0, jnp.float32)
        for i in range(NV):
            zbuf[pl.ds(i * L, L)] = zeros
            iref[pl.ds(i * L, L)] = lax.iota(jnp.int32, L) + i * L

        @pl.when(sid == 0)
        def _():
            pltpu.sync_copy(zbuf, sdeg)
            pltpu.sync_copy(zbuf, sga)
            pltpu.sync_copy(zbuf, sgb)
            pltpu.sync_copy(zbuf, sgc)

        for i in range(NV):
            deg[pl.ds(i * L, L)] = zeros

        def degbody(i, c):
            g = sid + i * L
            @pl.when(g < EV)
            def _():
                off = pl.multiple_of(g * L, L)
                plsc.addupdate_scatter(deg, [dst16(off)], ones)
            return c
        lax.fori_loop(0, GPT, degbody, 0)

        plsc.subcore_barrier()
        pltpu.sync_copy(deg, sdeg.at[iref], add=True)
        plsc.subcore_barrier()
        pltpu.sync_copy(sdeg, deg)

        for i in range(NV):
            s = pl.ds(i * L, L)
            idx = lax.iota(jnp.int32, L) + i * L
            real = idx < N
            dg = deg[s] + jnp.where(real, 1.0, 0.0)
            dinv[s] = _rsqrt16(jnp.where(real, dg, 1.0))

        w00 = _splat(pk, PBASE + 0)
        w01 = _splat(pk, PBASE + 1)
        w10 = _splat(pk, PBASE + 2)
        w11 = _splat(pk, PBASE + 3)
        w20 = _splat(pk, PBASE + 4)
        w21 = _splat(pk, PBASE + 5)
        for i in range(NV):
            s = pl.ds(i * L, L)
            base = jnp.minimum(lax.iota(jnp.int32, L) + i * L, N - 1) * 3
            f0 = plsc.load_gather(pk, [base])
            f1 = plsc.load_gather(pk, [base + 1])
            f2 = plsc.load_gather(pk, [base + 2])
            xa[s] = f0 * w00 + f1 * w10 + f2 * w20
            xb[s] = f0 * w01 + f1 * w11 + f2 * w21
            ga[s] = zeros
            gb[s] = zeros

        def e1(i, c):
            g = sid + i * L
            @pl.when(g < EV)
            def _():
                off = pl.multiple_of(g * L, L)
                sv = src16(off)
                dv = dst16(off)
                nm = plsc.load_gather(dinv, [sv]) * plsc.load_gather(dinv, [dv])
                normv[pl.ds(off, L)] = nm
                plsc.addupdate_scatter(ga, [dv], nm * plsc.load_gather(xa, [sv]))
                plsc.addupdate_scatter(gb, [dv], nm * plsc.load_gather(xb, [sv]))
            return c
        lax.fori_loop(0, GPT, e1, 0)

        pltpu.sync_copy(ga, sga.at[iref], add=True)
        pltpu.sync_copy(gb, sgb.at[iref], add=True)
        plsc.subcore_barrier()
        pltpu.sync_copy(sga, ga)
        pltpu.sync_copy(sgb, gb)

        b1a = _splat(pk, PBASE + 6)
        b1b = _splat(pk, PBASE + 7)
        w2a = _splat(pk, PBASE + 8)
        w2b = _splat(pk, PBASE + 9)
        for i in range(NV):
            s = pl.ds(i * L, L)
            dv2 = dinv[s] * dinv[s]
            va = jnp.maximum(ga[s] + dv2 * xa[s] + b1a, 0.0)
            vb = jnp.maximum(gb[s] + dv2 * xb[s] + b1b, 0.0)
            xc[s] = va * w2a + vb * w2b
            gc[s] = zeros

        def e2(i, c):
            g = sid + i * L
            @pl.when(g < EV)
            def _():
                off = pl.multiple_of(g * L, L)
                sv = src16(off)
                dv = dst16(off)
                plsc.addupdate_scatter(gc, [dv],
                                       normv[pl.ds(off, L)] * plsc.load_gather(xc, [sv]))
            return c
        lax.fori_loop(0, GPT, e2, 0)

        pltpu.sync_copy(gc, sgc.at[iref], add=True)
        plsc.subcore_barrier()

        @pl.when(sid == 0)
        def _():
            pltpu.sync_copy(sgc, gc)
            b2s = _splat(pk, PBASE + 10)
            for i in range(NV):
                s = pl.ds(i * L, L)
                idx = lax.iota(jnp.int32, L) + i * L
                dv2 = dinv[s] * dinv[s]
                hv = jnp.maximum(gc[s] + dv2 * xc[s] + b2s, 0.0)
                hout[pl.ds(i * L, L)] = jnp.where(idx < N, hv, 0.0)
            pltpu.sync_copy(hout, outh)


_SC_SCRATCH = [
    pltpu.VMEM((PKLEN,), jnp.float32),
    pltpu.VMEM((NP,), jnp.float32),
    pltpu.VMEM((NP,), jnp.float32),
    pltpu.VMEM((NP,), jnp.float32),
    pltpu.VMEM((NP,), jnp.float32),
    pltpu.VMEM((NP,), jnp.float32),
    pltpu.VMEM((NP,), jnp.float32),
    pltpu.VMEM((NP,), jnp.float32),
    pltpu.VMEM((NP,), jnp.float32),
    pltpu.VMEM((E,), jnp.float32),
    pltpu.VMEM((NP,), jnp.float32),
    pltpu.VMEM((NP,), jnp.float32),
    pltpu.VMEM((NP,), jnp.int32),
    pltpu.VMEM_SHARED((NP,), jnp.float32),
    pltpu.VMEM_SHARED((NP,), jnp.float32),
    pltpu.VMEM_SHARED((NP,), jnp.float32),
    pltpu.VMEM_SHARED((NP,), jnp.float32),
]


def _sc_gcn(pk):
    fn = functools.partial(
        pl.kernel,
        out_type=jax.ShapeDtypeStruct((NP,), jnp.float32),
        mesh=plsc.VectorSubcoreMesh(core_axis_name="c", subcore_axis_name="s"),
        scratch_types=_SC_SCRATCH,
        compiler_params=pltpu.CompilerParams(needs_layout_passes=False),
    )(_gcn_body)
    return fn(pk)


KBLK = 256
NBLK = 1024 // KBLK


def _tc_body(h_ref, w1_ref, b1_ref, w2_ref, b2_ref, w3_ref, b3_ref,
             o_ref, h2_ref):
    i = pl.program_id(0)

    @pl.when(i == 0)
    def _():
        x = jnp.transpose(h_ref[...][None, :], (1, 0))
        w1 = jnp.concatenate(
            [w1_ref[...], jnp.zeros((NP - N, 512), jnp.float32)])
        h1 = jnp.sum(x * w1, axis=0, keepdims=True)
        h1 = jnp.maximum(h1 + b1_ref[...][None, :], 0.0)
        h1c = jnp.transpose(h1, (1, 0))
        h2 = jnp.sum(h1c * w2_ref[...], axis=0, keepdims=True)
        h2 = jnp.maximum(h2 + b2_ref[...][None, :], 0.0)
        h2_ref[...] = jnp.transpose(h2, (1, 0))

    acc = h2_ref[pl.ds(i * KBLK, KBLK), :] * w3_ref[...]
    for sz in (128, 64, 32, 16, 8):
        acc = acc[:sz] + acc[sz:]
    part = jnp.sum(acc, axis=0)

    @pl.when(i == 0)
    def _():
        o_ref[...] = part + b3_ref[...]

    @pl.when(i > 0)
    def _():
        o_ref[...] += part


def _tc_mlp(h, w1, b1, w2, b2, w3, b3):
    return pl.pallas_call(
        _tc_body,
        grid=(NBLK,),
        in_specs=[
            pl.BlockSpec((NP,), lambda j: (0,)),
            pl.BlockSpec((N, 512), lambda j: (0, 0)),
            pl.BlockSpec((512,), lambda j: (0,)),
            pl.BlockSpec((512, 1024), lambda j: (0, 0)),
            pl.BlockSpec((1024,), lambda j: (0,)),
            pl.BlockSpec((KBLK, 6400), lambda j: (j, 0)),
            pl.BlockSpec((6400,), lambda j: (0,)),
        ],
        out_specs=pl.BlockSpec((6400,), lambda j: (0,)),
        out_shape=jax.ShapeDtypeStruct((6400,), jnp.float32),
        scratch_shapes=[pltpu.VMEM((1024, 1), jnp.float32)],
        compiler_params=pltpu.CompilerParams(
            dimension_semantics=("arbitrary",)),
    )(h, w1, b1, w2, b2, w3, b3)


def kernel(feature, edge_index, W1, b1, W2, b2, Wfc1, bfc1, Wfc2, bfc2, Wfc, bfc):
    pk = jnp.concatenate(
        [feature.reshape(-1), jnp.zeros((PBASE - 3 * N,), jnp.float32),
         W1.reshape(-1), b1, W2.reshape(-1), b2,
         jnp.zeros((EBASE - PBASE - 11,), jnp.float32),
         lax.bitcast_convert_type(edge_index.reshape(-1), jnp.float32)])
    h96 = _sc_gcn(pk)
    return _tc_mlp(h96, Wfc1, bfc1, Wfc2, bfc2, Wfc, bfc)


# speedup vs baseline: 1.3342x; 1.0047x over previous
import functools

import jax
import jax.numpy as jnp
from jax import lax
from jax.experimental import pallas as pl
from jax.experimental.pallas import tpu as pltpu
from jax.experimental.pallas import tpu_sc as plsc

N = 94
NP = 96
E = 3008
L = 16
EV = E // L
NV = NP // L
UNROLL = 4
PBASE = 288
EBASE = 304
PKLEN = EBASE + 2 * E


def _splat(ref, k):
    return plsc.load_gather(ref, [jnp.full((L,), k, jnp.int32)])


def _rsqrt16(x):
    i = plsc.bitcast(x, jnp.int32)
    i = jnp.int32(0x5F3759DF) - (i >> 1)
    y = plsc.bitcast(i, jnp.float32)
    for _ in range(3):
        y = y * (1.5 - 0.5 * x * y * y)
    return y


GPT = 12


def _gcn_body(pkh, outh,
              pk,
              deg, dinv, xa, xb, ga, gb, xc, gc, normv, hout, zbuf, iref,
              sdeg, sga, sgb, sgc):
    cid = lax.axis_index("c")
    sid = lax.axis_index("s")

    @pl.when(cid == 0)
    def _():
        pltpu.sync_copy(pkh, pk)

        def src16(off):
            return plsc.bitcast(pk[pl.ds(EBASE + off, L)], jnp.int32)

        def dst16(off):
            return plsc.bitcast(pk[pl.ds(EBASE + E + off, L)], jnp.int32)

        zeros = jnp.zeros((L,), jnp.float32)
        ones = jnp.full((L,), 1.0, jnp.float32)
        for i in range(NV):
            zbuf[pl.ds(i * L, L)] = zeros
            iref[pl.ds(i * L, L)] = lax.iota(jnp.int32, L) + i * L

        @pl.when(sid == 0)
        def _():
            pltpu.sync_copy(zbuf, sdeg)
            pltpu.sync_copy(zbuf, sga)
            pltpu.sync_copy(zbuf, sgb)
            pltpu.sync_copy(zbuf, sgc)

        for i in range(NV):
            deg[pl.ds(i * L, L)] = zeros

        def degbody(i, c):
            g = sid + i * L
            @pl.when(g < EV)
            def _():
                off = pl.multiple_of(g * L, L)
                plsc.addupdate_scatter(deg, [dst16(off)], ones)
            return c
        lax.fori_loop(0, GPT, degbody, 0)

        plsc.subcore_barrier()
        pltpu.sync_copy(deg, sdeg.at[iref], add=True)
        plsc.subcore_barrier()
        pltpu.sync_copy(sdeg, deg)

        for i in range(NV):
            s = pl.ds(i * L, L)
            idx = lax.iota(jnp.int32, L) + i * L
            real = idx < N
            dg = deg[s] + jnp.where(real, 1.0, 0.0)
            dinv[s] = _rsqrt16(jnp.where(real, dg, 1.0))

        w00 = _splat(pk, PBASE + 0)
        w01 = _splat(pk, PBASE + 1)
        w10 = _splat(pk, PBASE + 2)
        w11 = _splat(pk, PBASE + 3)
        w20 = _splat(pk, PBASE + 4)
        w21 = _splat(pk, PBASE + 5)
        for i in range(NV):
            s = pl.ds(i * L, L)
            base = jnp.minimum(lax.iota(jnp.int32, L) + i * L, N - 1) * 3
            f0 = plsc.load_gather(pk, [base])
            f1 = plsc.load_gather(pk, [base + 1])
            f2 = plsc.load_gather(pk, [base + 2])
            xa[s] = f0 * w00 + f1 * w10 + f2 * w20
            xb[s] = f0 * w01 + f1 * w11 + f2 * w21
            ga[s] = zeros
            gb[s] = zeros

        def e1(i, c):
            g = sid + i * L
            @pl.when(g < EV)
            def _():
                off = pl.multiple_of(g * L, L)
                sv = src16(off)
                dv = dst16(off)
                nm = plsc.load_gather(dinv, [sv]) * plsc.load_gather(dinv, [dv])
                normv[pl.ds(off, L)] = nm
                plsc.addupdate_scatter(ga, [dv], nm * plsc.load_gather(xa, [sv]))
                plsc.addupdate_scatter(gb, [dv], nm * plsc.load_gather(xb, [sv]))
            return c
        lax.fori_loop(0, GPT, e1, 0)

        pltpu.sync_copy(ga, sga.at[iref], add=True)
        pltpu.sync_copy(gb, sgb.at[iref], add=True)
        plsc.subcore_barrier()
        pltpu.sync_copy(sga, ga)
        pltpu.sync_copy(sgb, gb)

        b1a = _splat(pk, PBASE + 6)
        b1b = _splat(pk, PBASE + 7)
        w2a = _splat(pk, PBASE + 8)
        w2b = _splat(pk, PBASE + 9)
        for i in range(NV):
            s = pl.ds(i * L, L)
            dv2 = dinv[s] * dinv[s]
            va = jnp.maximum(ga[s] + dv2 * xa[s] + b1a, 0.0)
            vb = jnp.maximum(gb[s] + dv2 * xb[s] + b1b, 0.0)
            xc[s] = va * w2a + vb * w2b
            gc[s] = zeros

        def e2(i, c):
            g = sid + i * L
            @pl.when(g < EV)
            def _():
                off = pl.multiple_of(g * L, L)
                sv = src16(off)
                dv = dst16(off)
                plsc.addupdate_scatter(gc, [dv],
                                       normv[pl.ds(off, L)] * plsc.load_gather(xc, [sv]))
            return c
        lax.fori_loop(0, GPT, e2, 0)

        pltpu.sync_copy(gc, sgc.at[iref], add=True)
        plsc.subcore_barrier()

        @pl.when(sid == 0)
        def _():
            pltpu.sync_copy(sgc, gc)
            b2s = _splat(pk, PBASE + 10)
            for i in range(NV):
                s = pl.ds(i * L, L)
                idx = lax.iota(jnp.int32, L) + i * L
                dv2 = dinv[s] * dinv[s]
                hv = jnp.maximum(gc[s] + dv2 * xc[s] + b2s, 0.0)
                hout[pl.ds(i * L, L)] = jnp.where(idx < N, hv, 0.0)
            pltpu.sync_copy(hout, outh)


_SC_SCRATCH = [
    pltpu.VMEM((PKLEN,), jnp.float32),
    pltpu.VMEM((NP,), jnp.float32),
    pltpu.VMEM((NP,), jnp.float32),
    pltpu.VMEM((NP,), jnp.float32),
    pltpu.VMEM((NP,), jnp.float32),
    pltpu.VMEM((NP,), jnp.float32),
    pltpu.VMEM((NP,), jnp.float32),
    pltpu.VMEM((NP,), jnp.float32),
    pltpu.VMEM((NP,), jnp.float32),
    pltpu.VMEM((E,), jnp.float32),
    pltpu.VMEM((NP,), jnp.float32),
    pltpu.VMEM((NP,), jnp.float32),
    pltpu.VMEM((NP,), jnp.int32),
    pltpu.VMEM_SHARED((NP,), jnp.float32),
    pltpu.VMEM_SHARED((NP,), jnp.float32),
    pltpu.VMEM_SHARED((NP,), jnp.float32),
    pltpu.VMEM_SHARED((NP,), jnp.float32),
]


def _sc_gcn(pk):
    fn = functools.partial(
        pl.kernel,
        out_type=jax.ShapeDtypeStruct((NP,), jnp.float32),
        mesh=plsc.VectorSubcoreMesh(core_axis_name="c", subcore_axis_name="s"),
        scratch_types=_SC_SCRATCH,
        compiler_params=pltpu.CompilerParams(needs_layout_passes=False),
    )(_gcn_body)
    return fn(pk)


KBLK = 512
NBLK = 1024 // KBLK


def _tc_body(h_ref, w1_ref, b1_ref, w2_ref, b2_ref, w3_ref, b3_ref,
             o_ref, h2_ref):
    i = pl.program_id(0)

    @pl.when(i == 0)
    def _():
        x = jnp.transpose(h_ref[...][None, :], (1, 0))
        w1 = jnp.concatenate(
            [w1_ref[...], jnp.zeros((NP - N, 512), jnp.float32)])
        h1 = jnp.sum(x * w1, axis=0, keepdims=True)
        h1 = jnp.maximum(h1 + b1_ref[...][None, :], 0.0)
        h1c = jnp.transpose(h1, (1, 0))
        h2 = jnp.sum(h1c * w2_ref[...], axis=0, keepdims=True)
        h2 = jnp.maximum(h2 + b2_ref[...][None, :], 0.0)
        h2_ref[...] = jnp.transpose(h2, (1, 0))

    acc = h2_ref[pl.ds(i * KBLK, KBLK), :] * w3_ref[...]
    for sz in (256, 128, 64, 32, 16, 8):
        acc = acc[:sz] + acc[sz:]
    part = jnp.sum(acc, axis=0)

    @pl.when(i == 0)
    def _():
        o_ref[...] = part + b3_ref[...]

    @pl.when(i > 0)
    def _():
        o_ref[...] += part


def _tc_mlp(h, w1, b1, w2, b2, w3, b3):
    return pl.pallas_call(
        _tc_body,
        grid=(NBLK,),
        in_specs=[
            pl.BlockSpec((NP,), lambda j: (0,)),
            pl.BlockSpec((N, 512), lambda j: (0, 0)),
            pl.BlockSpec((512,), lambda j: (0,)),
            pl.BlockSpec((512, 1024), lambda j: (0, 0)),
            pl.BlockSpec((1024,), lambda j: (0,)),
            pl.BlockSpec((KBLK, 6400), lambda j: (j, 0)),
            pl.BlockSpec((6400,), lambda j: (0,)),
        ],
        out_specs=pl.BlockSpec((6400,), lambda j: (0,)),
        out_shape=jax.ShapeDtypeStruct((6400,), jnp.float32),
        scratch_shapes=[pltpu.VMEM((1024, 1), jnp.float32)],
        compiler_params=pltpu.CompilerParams(
            dimension_semantics=("arbitrary",)),
    )(h, w1, b1, w2, b2, w3, b3)


def kernel(feature, edge_index, W1, b1, W2, b2, Wfc1, bfc1, Wfc2, bfc2, Wfc, bfc):
    pk = jnp.concatenate(
        [feature.reshape(-1), jnp.zeros((PBASE - 3 * N,), jnp.float32),
         W1.reshape(-1), b1, W2.reshape(-1), b2,
         jnp.zeros((EBASE - PBASE - 11,), jnp.float32),
         lax.bitcast_convert_type(edge_index.reshape(-1), jnp.float32)])
    h96 = _sc_gcn(pk)
    return _tc_mlp(h96, Wfc1, bfc1, Wfc2, bfc2, Wfc, bfc)
